# baseline (device time: 47360 ns/iter reference)
import jax
import jax.numpy as jnp
from jax import lax
from jax.experimental import pallas as pl
from jax.experimental.pallas import tpu as pltpu

_CompilerParams = (
    pltpu.CompilerParams
    if hasattr(pltpu, "CompilerParams")
    else pltpu.TPUCompilerParams
)

_MESH = pl.DeviceIdType.MESH
NC = 8
_LOG2E = 1.4426950408889634


def kernel(Q, K, V):
    b, s_loc, h, d = Q.shape
    hd = h * d
    hp = h // 2
    nq = NC // 2
    cs = s_loc // nq
    scale = d**-0.5 * _LOG2E

    qb = (Q * scale).astype(jnp.bfloat16).reshape(b, s_loc, hd)
    kb = K.astype(jnp.bfloat16).reshape(b, s_loc, hd)
    vb = V.astype(jnp.bfloat16).reshape(b, s_loc, hd)

    def body(q_ref, k_ref, v_ref, o_ref, rkv, racc,
             yks, ykr, yvs, yvr, xs, xr):
        my_x = lax.axis_index("x")
        my_y = lax.axis_index("y")
        ynbr = (my_x, 1 - my_y)
        xnbr = (1 - my_x, my_y)
        mb = 2 * my_x
        ob = 2 * (1 - my_x)

        bsem = pltpu.get_barrier_semaphore()
        pl.semaphore_signal(bsem, inc=1, device_id=ynbr, device_id_type=_MESH)
        pl.semaphore_signal(bsem, inc=1, device_id=xnbr, device_id_type=_MESH)
        pl.semaphore_wait(bsem, 2)

        ydk, ydv, xd = [], [], []
        for c in range(NC):
            bsl = pl.ds(mb + c // nq, 1)
            ssl = pl.ds((c % nq) * cs, cs)
            ydk.append(
                pltpu.make_async_remote_copy(
                    src_ref=k_ref.at[bsl, ssl],
                    dst_ref=rkv.at[bsl, ssl, pl.ds(0, hd)],
                    send_sem=yks.at[c],
                    recv_sem=ykr.at[c],
                    device_id=ynbr,
                    device_id_type=_MESH,
                )
            )
            ydv.append(
                pltpu.make_async_remote_copy(
                    src_ref=v_ref.at[bsl, ssl],
                    dst_ref=rkv.at[bsl, ssl, pl.ds(hd, hd)],
                    send_sem=yvs.at[c],
                    recv_sem=yvr.at[c],
                    device_id=ynbr,
                    device_id_type=_MESH,
                )
            )
            xd.append(
                pltpu.make_async_remote_copy(
                    src_ref=rkv.at[bsl, ssl],
                    dst_ref=rkv.at[bsl, ssl],
                    send_sem=xs.at[c],
                    recv_sem=xr.at[c],
                    device_id=xnbr,
                    device_id_type=_MESH,
                )
            )
        for c in range(NC):
            ydk[c].start()
            ydv[c].start()

        ones = jnp.ones((s_loc, d), jnp.bfloat16)
        dn = (((1,), (1,)), ((), ()))
        dn2 = (((1,), (0,)), ((), ()))

        def partial(qh, kk, vv):
            s = lax.dot_general(qh, kk, dn, preferred_element_type=jnp.float32)
            p = jnp.exp2(s).astype(jnp.bfloat16)
            vx = jnp.concatenate([vv, ones], axis=1)
            return lax.dot_general(p, vx, dn2,
                                   preferred_element_type=jnp.float32)

        def local_block(bi, hi):
            qp = q_ref[bi, :, hi * 2 * d:(hi + 1) * 2 * d]
            kp = k_ref[bi, :, hi * 2 * d:(hi + 1) * 2 * d]
            vp = v_ref[bi, :, hi * 2 * d:(hi + 1) * 2 * d]
            rs = [partial(qp[:, j * d:(j + 1) * d],
                          kp[:, j * d:(j + 1) * d],
                          vp[:, j * d:(j + 1) * d]) for j in (0, 1)]
            racc[bi, hi] = jnp.concatenate(rs, axis=1)

        def remote_block(bi, hi):
            qp = q_ref[bi, :, hi * 2 * d:(hi + 1) * 2 * d]
            kv_pair = rkv[bi, :, :]
            outs = []
            for j in (0, 1):
                off = hi * 2 * d + j * d
                rr = partial(qp[:, j * d:(j + 1) * d],
                             kv_pair[:, off:off + d],
                             kv_pair[:, hd + off:hd + off + d])
                rt = racc[bi, hi, :, j * 2 * d:(j + 1) * 2 * d] + rr
                outs.append(rt[:, :d] / rt[:, d:])
            o_ref[bi, :, hi * 2 * d:(hi + 1) * 2 * d] = (
                jnp.concatenate(outs, axis=1)
            )

        blocks = [(bi, hi) for bi in range(b) for hi in range(hp)]
        for c in range(nq):
            for bi, hi in blocks[c * 4:(c + 1) * 4]:
                local_block(bi, hi)
            ydk[c].wait_recv()
            ydv[c].wait_recv()
            xd[c].start()
        for c in range(nq, NC):
            for bi, hi in blocks[c * 4:(c + 1) * 4]:
                local_block(bi, hi)
            for hi in (2 * (c - nq), 2 * (c - nq) + 1):
                remote_block(mb, hi)
            ydk[c].wait_recv()
            ydv[c].wait_recv()
            xd[c].start()
        for hi in range(hp):
            remote_block(mb + 1, hi)
        for c in range(NC):
            ydk[c].wait_send()
            ydv[c].wait_send()
        for c in range(NC):
            xd[c].wait_send()
            xd[c].wait_recv()
        for bj in (0, 1):
            for hi in range(hp):
                remote_block(ob + bj, hi)

    out = pl.pallas_call(
        body,
        out_shape=jax.ShapeDtypeStruct((b, s_loc, hd), jnp.float32),
        in_specs=[pl.BlockSpec(memory_space=pltpu.VMEM)] * 3,
        out_specs=pl.BlockSpec(memory_space=pltpu.VMEM),
        scratch_shapes=[
            pltpu.VMEM((b, s_loc, 2 * hd), jnp.bfloat16),
            pltpu.VMEM((b, hp, s_loc, 4 * d), jnp.float32),
            pltpu.SemaphoreType.DMA((NC,)),
            pltpu.SemaphoreType.DMA((NC,)),
            pltpu.SemaphoreType.DMA((NC,)),
            pltpu.SemaphoreType.DMA((NC,)),
            pltpu.SemaphoreType.DMA((NC,)),
            pltpu.SemaphoreType.DMA((NC,)),
        ],
        compiler_params=_CompilerParams(collective_id=0),
    )(qb, kb, vb)

    return out.reshape(b, s_loc, h, d)
